# Initial kernel scaffold; baseline (speedup 1.0000x reference)
import jax
import jax.numpy as jnp
from jax.experimental import pallas as pl


def kernel(q_curve_xyz, b_curve_xyz):
    raise NotImplementedError("write your pallas kernel here")


# speedup vs baseline: 11.2917x; 11.2917x over previous
import jax
import jax.numpy as jnp
from jax.experimental import pallas as pl

T = 32
N = 512
M = 2048
TOPK_TIME = 8
KNN = 16
N_TILE = 128
M_TILE = 256


def _dist_kernel(q_ref, b_ref, dist_ref):
    q = q_ref[...]
    b = b_ref[...]
    neg_inf = jnp.float32(-jnp.inf)
    acc = [jnp.full((N_TILE, M_TILE), neg_inf, dtype=jnp.float32)
           for _ in range(TOPK_TIME)]
    for t in range(T):
        qt = q[:, t, :]
        bt = b[t]
        q2 = jnp.sum(qt * qt, axis=1, keepdims=True)
        b2 = jnp.sum(bt * bt, axis=0, keepdims=True)
        cross = jax.lax.dot_general(
            qt, bt, (((1,), (0,)), ((), ())),
            preferred_element_type=jnp.float32)
        d2 = q2 + b2 - 2.0 * cross
        v = d2
        for i in range(TOPK_TIME):
            hi = jnp.maximum(acc[i], v)
            v = jnp.minimum(acc[i], v)
            acc[i] = hi
    d2_sel = acc[TOPK_TIME - 1]
    dist_ref[...] = jnp.sqrt(jnp.clip(d2_sel, 0.0, None) + 1e-12)


def _topk_kernel(dist_ref, knn_dist_ref, knn_ind_ref):
    work = dist_ref[...]
    iota = jax.lax.broadcasted_iota(jnp.int32, (N_TILE, M), 1)
    big_i = jnp.int32(M)
    pos_inf = jnp.float32(jnp.inf)
    for k in range(KNN):
        mv = jnp.min(work, axis=1, keepdims=True)
        idx = jnp.min(jnp.where(work == mv, iota, big_i), axis=1,
                      keepdims=True)
        knn_dist_ref[:, k] = mv[:, 0]
        knn_ind_ref[:, k] = idx[:, 0]
        work = jnp.where(iota == idx, pos_inf, work)


@jax.jit
def kernel(q_curve_xyz, b_curve_xyz):
    q_r = jnp.transpose(q_curve_xyz, (1, 0, 2))
    b_r = jnp.transpose(b_curve_xyz, (0, 2, 1))

    dist = pl.pallas_call(
        _dist_kernel,
        grid=(N // N_TILE, M // M_TILE),
        in_specs=[
            pl.BlockSpec((N_TILE, T, 3), lambda i, j: (i, 0, 0)),
            pl.BlockSpec((T, 3, M_TILE), lambda i, j: (0, 0, j)),
        ],
        out_specs=pl.BlockSpec((N_TILE, M_TILE), lambda i, j: (i, j)),
        out_shape=jax.ShapeDtypeStruct((N, M), jnp.float32),
    )(q_r, b_r)

    knn_dist, knn_ind = pl.pallas_call(
        _topk_kernel,
        grid=(N // N_TILE,),
        in_specs=[pl.BlockSpec((N_TILE, M), lambda i: (i, 0))],
        out_specs=[
            pl.BlockSpec((N_TILE, KNN), lambda i: (i, 0)),
            pl.BlockSpec((N_TILE, KNN), lambda i: (i, 0)),
        ],
        out_shape=[
            jax.ShapeDtypeStruct((N, KNN), jnp.float32),
            jax.ShapeDtypeStruct((N, KNN), jnp.int32),
        ],
    )(dist)
    return (knn_dist, knn_ind)


# speedup vs baseline: 13.4800x; 1.1938x over previous
import jax
import jax.numpy as jnp
from jax.experimental import pallas as pl

T = 32
N = 512
M = 2048
TOPK_TIME = 8
KNN = 16
N_TILE = 128
M_TILE = 256


_SORT8_NET = [(0, 1), (2, 3), (4, 5), (6, 7),
              (0, 2), (1, 3), (4, 6), (5, 7),
              (1, 2), (5, 6), (0, 4), (3, 7),
              (1, 5), (2, 6),
              (1, 4), (3, 6),
              (2, 4), (3, 5),
              (3, 4)]

_BITONIC8_NET = [(0, 4), (1, 5), (2, 6), (3, 7),
                 (0, 2), (1, 3), (4, 6), (5, 7),
                 (0, 1), (2, 3), (4, 5), (6, 7)]


def _apply_net(vs, net):
    vs = list(vs)
    for i, j in net:
        hi = jnp.maximum(vs[i], vs[j])
        lo = jnp.minimum(vs[i], vs[j])
        vs[i], vs[j] = hi, lo
    return vs


def _dist_kernel(q_ref, b_ref, dist_ref):
    q = q_ref[...]
    b = b_ref[...]
    run = None
    n_groups = T // TOPK_TIME
    for g in range(n_groups):
        d2s = []
        for tt in range(TOPK_TIME):
            t = g * TOPK_TIME + tt
            qt = q[:, t, :]
            bt = b[t]
            q2 = jnp.sum(qt * qt, axis=1, keepdims=True)
            b2 = jnp.sum(bt * bt, axis=0, keepdims=True)
            cross = jax.lax.dot_general(
                qt, bt, (((1,), (0,)), ((), ())),
                preferred_element_type=jnp.float32)
            d2s.append(q2 + b2 - 2.0 * cross)
        s = _apply_net(d2s, _SORT8_NET)
        if run is None:
            run = s
        elif g < n_groups - 1:
            tops = [jnp.maximum(run[i], s[TOPK_TIME - 1 - i])
                    for i in range(TOPK_TIME)]
            run = _apply_net(tops, _BITONIC8_NET)
        else:
            tops = [jnp.maximum(run[i], s[TOPK_TIME - 1 - i])
                    for i in range(TOPK_TIME)]
            d2_sel = tops[0]
            for i in range(1, TOPK_TIME):
                d2_sel = jnp.minimum(d2_sel, tops[i])
    dist_ref[...] = jnp.sqrt(jnp.clip(d2_sel, 0.0, None) + 1e-12)


def _topk_kernel(dist_ref, knn_dist_ref, knn_ind_ref):
    work = dist_ref[...]
    iota = jax.lax.broadcasted_iota(jnp.int32, (N_TILE, M), 1)
    big_i = jnp.int32(M)
    pos_inf = jnp.float32(jnp.inf)
    for k in range(KNN):
        mv = jnp.min(work, axis=1, keepdims=True)
        idx = jnp.min(jnp.where(work == mv, iota, big_i), axis=1,
                      keepdims=True)
        knn_dist_ref[:, k] = mv[:, 0]
        knn_ind_ref[:, k] = idx[:, 0]
        work = jnp.where(iota == idx, pos_inf, work)


@jax.jit
def kernel(q_curve_xyz, b_curve_xyz):
    q_r = jnp.transpose(q_curve_xyz, (1, 0, 2))
    b_r = jnp.transpose(b_curve_xyz, (0, 2, 1))

    dist = pl.pallas_call(
        _dist_kernel,
        grid=(N // N_TILE, M // M_TILE),
        in_specs=[
            pl.BlockSpec((N_TILE, T, 3), lambda i, j: (i, 0, 0)),
            pl.BlockSpec((T, 3, M_TILE), lambda i, j: (0, 0, j)),
        ],
        out_specs=pl.BlockSpec((N_TILE, M_TILE), lambda i, j: (i, j)),
        out_shape=jax.ShapeDtypeStruct((N, M), jnp.float32),
    )(q_r, b_r)

    knn_dist, knn_ind = pl.pallas_call(
        _topk_kernel,
        grid=(N // N_TILE,),
        in_specs=[pl.BlockSpec((N_TILE, M), lambda i: (i, 0))],
        out_specs=[
            pl.BlockSpec((N_TILE, KNN), lambda i: (i, 0)),
            pl.BlockSpec((N_TILE, KNN), lambda i: (i, 0)),
        ],
        out_shape=[
            jax.ShapeDtypeStruct((N, KNN), jnp.float32),
            jax.ShapeDtypeStruct((N, KNN), jnp.int32),
        ],
    )(dist)
    return (knn_dist, knn_ind)


# speedup vs baseline: 13.6552x; 1.0130x over previous
import jax
import jax.numpy as jnp
from jax.experimental import pallas as pl

T = 32
N = 512
M = 2048
TOPK_TIME = 8
KNN = 16
N_TILE = 128
M_TILE = 256


_SORT8_NET = [(0, 1), (2, 3), (4, 5), (6, 7),
              (0, 2), (1, 3), (4, 6), (5, 7),
              (1, 2), (5, 6), (0, 4), (3, 7),
              (1, 5), (2, 6),
              (1, 4), (3, 6),
              (2, 4), (3, 5),
              (3, 4)]

_BITONIC8_NET = [(0, 4), (1, 5), (2, 6), (3, 7),
                 (0, 2), (1, 3), (4, 6), (5, 7),
                 (0, 1), (2, 3), (4, 5), (6, 7)]


def _apply_net(vs, net):
    vs = list(vs)
    for i, j in net:
        hi = jnp.maximum(vs[i], vs[j])
        lo = jnp.minimum(vs[i], vs[j])
        vs[i], vs[j] = hi, lo
    return vs


def _dist_kernel(q_ref, b_ref, dist_ref):
    q = q_ref[...]
    b = b_ref[...]
    run = None
    n_groups = T // TOPK_TIME
    for g in range(n_groups):
        d2s = []
        for tt in range(TOPK_TIME):
            t = g * TOPK_TIME + tt
            qt = q[:, t, :]
            bt = b[t]
            q2 = 0.25 * jnp.sum(qt * qt, axis=1, keepdims=True)
            b2 = jnp.sum(bt * bt, axis=0, keepdims=True)
            cross = jax.lax.dot_general(
                qt, bt, (((1,), (0,)), ((), ())),
                preferred_element_type=jnp.float32)
            d2s.append((q2 + b2) + cross)
        s = _apply_net(d2s, _SORT8_NET)
        if run is None:
            run = s
        elif g < n_groups - 1:
            tops = [jnp.maximum(run[i], s[TOPK_TIME - 1 - i])
                    for i in range(TOPK_TIME)]
            run = _apply_net(tops, _BITONIC8_NET)
        else:
            tops = [jnp.maximum(run[i], s[TOPK_TIME - 1 - i])
                    for i in range(TOPK_TIME)]
            d2_sel = tops[0]
            for i in range(1, TOPK_TIME):
                d2_sel = jnp.minimum(d2_sel, tops[i])
    dist_ref[...] = jnp.sqrt(jnp.clip(d2_sel, 0.0, None) + 1e-12)


def _topk_kernel(dist_ref, knn_dist_ref, knn_ind_ref):
    work = dist_ref[...]
    iota = jax.lax.broadcasted_iota(jnp.int32, (N_TILE, M), 1)
    big_i = jnp.int32(M)
    pos_inf = jnp.float32(jnp.inf)
    for k in range(KNN):
        mv = jnp.min(work, axis=1, keepdims=True)
        idx = jnp.min(jnp.where(work == mv, iota, big_i), axis=1,
                      keepdims=True)
        knn_dist_ref[:, k] = mv[:, 0]
        knn_ind_ref[:, k] = idx[:, 0]
        work = jnp.where(iota == idx, pos_inf, work)


@jax.jit
def kernel(q_curve_xyz, b_curve_xyz):
    q_r = -2.0 * jnp.transpose(q_curve_xyz, (1, 0, 2))
    b_r = jnp.transpose(b_curve_xyz, (0, 2, 1))

    dist = pl.pallas_call(
        _dist_kernel,
        grid=(N // N_TILE, M // M_TILE),
        in_specs=[
            pl.BlockSpec((N_TILE, T, 3), lambda i, j: (i, 0, 0)),
            pl.BlockSpec((T, 3, M_TILE), lambda i, j: (0, 0, j)),
        ],
        out_specs=pl.BlockSpec((N_TILE, M_TILE), lambda i, j: (i, j)),
        out_shape=jax.ShapeDtypeStruct((N, M), jnp.float32),
    )(q_r, b_r)

    knn_dist, knn_ind = pl.pallas_call(
        _topk_kernel,
        grid=(N // N_TILE,),
        in_specs=[pl.BlockSpec((N_TILE, M), lambda i: (i, 0))],
        out_specs=[
            pl.BlockSpec((N_TILE, KNN), lambda i: (i, 0)),
            pl.BlockSpec((N_TILE, KNN), lambda i: (i, 0)),
        ],
        out_shape=[
            jax.ShapeDtypeStruct((N, KNN), jnp.float32),
            jax.ShapeDtypeStruct((N, KNN), jnp.int32),
        ],
    )(dist)
    return (knn_dist, knn_ind)


# speedup vs baseline: 16.8775x; 1.2360x over previous
import jax
import jax.numpy as jnp
from jax.experimental import pallas as pl

T = 32
N = 512
M = 2048
TOPK_TIME = 8
KNN = 16
N_TILE = 128
M_TILE = 512


_SORT8_NET = [(0, 1), (2, 3), (4, 5), (6, 7),
              (0, 2), (1, 3), (4, 6), (5, 7),
              (1, 2), (5, 6), (0, 4), (3, 7),
              (1, 5), (2, 6),
              (1, 4), (3, 6),
              (2, 4), (3, 5),
              (3, 4)]

_BITONIC8_NET = [(0, 4), (1, 5), (2, 6), (3, 7),
                 (0, 2), (1, 3), (4, 6), (5, 7),
                 (0, 1), (2, 3), (4, 5), (6, 7)]


def _apply_net(vs, net):
    vs = list(vs)
    for i, j in net:
        hi = jnp.maximum(vs[i], vs[j])
        lo = jnp.minimum(vs[i], vs[j])
        vs[i], vs[j] = hi, lo
    return vs


def _dist_kernel(q_ref, b_ref, dist_ref):
    q = q_ref[...]
    b = b_ref[...]
    run = None
    n_groups = T // TOPK_TIME
    for g in range(n_groups):
        d2s = []
        for tt in range(TOPK_TIME):
            t = g * TOPK_TIME + tt
            qt = q[:, t, :]
            bt = b[t]
            q2 = 0.25 * jnp.sum(qt * qt, axis=1, keepdims=True)
            b2 = jnp.sum(bt * bt, axis=0, keepdims=True)
            cross = jax.lax.dot_general(
                qt, bt, (((1,), (0,)), ((), ())),
                preferred_element_type=jnp.float32)
            d2s.append((q2 + b2) + cross)
        s = _apply_net(d2s, _SORT8_NET)
        if run is None:
            run = s
        elif g < n_groups - 1:
            tops = [jnp.maximum(run[i], s[TOPK_TIME - 1 - i])
                    for i in range(TOPK_TIME)]
            run = _apply_net(tops, _BITONIC8_NET)
        else:
            tops = [jnp.maximum(run[i], s[TOPK_TIME - 1 - i])
                    for i in range(TOPK_TIME)]
            d2_sel = tops[0]
            for i in range(1, TOPK_TIME):
                d2_sel = jnp.minimum(d2_sel, tops[i])
    dist_ref[...] = jnp.sqrt(jnp.clip(d2_sel, 0.0, None) + 1e-12)


def _topk_kernel(dist_ref, knn_dist_ref, knn_ind_ref):
    work = dist_ref[...]
    iota = jax.lax.broadcasted_iota(jnp.int32, (N_TILE, M), 1)
    big_i = jnp.int32(M)
    pos_inf = jnp.float32(jnp.inf)
    for k in range(KNN):
        mv = jnp.min(work, axis=1, keepdims=True)
        idx = jnp.min(jnp.where(work == mv, iota, big_i), axis=1,
                      keepdims=True)
        knn_dist_ref[:, k] = mv[:, 0]
        knn_ind_ref[:, k] = idx[:, 0]
        work = jnp.where(iota == idx, pos_inf, work)


@jax.jit
def kernel(q_curve_xyz, b_curve_xyz):
    q_r = -2.0 * jnp.transpose(q_curve_xyz, (1, 0, 2))
    b_r = jnp.transpose(b_curve_xyz, (0, 2, 1))

    dist = pl.pallas_call(
        _dist_kernel,
        grid=(N // N_TILE, M // M_TILE),
        in_specs=[
            pl.BlockSpec((N_TILE, T, 3), lambda i, j: (i, 0, 0)),
            pl.BlockSpec((T, 3, M_TILE), lambda i, j: (0, 0, j)),
        ],
        out_specs=pl.BlockSpec((N_TILE, M_TILE), lambda i, j: (i, j)),
        out_shape=jax.ShapeDtypeStruct((N, M), jnp.float32),
    )(q_r, b_r)

    knn_dist, knn_ind = pl.pallas_call(
        _topk_kernel,
        grid=(N // N_TILE,),
        in_specs=[pl.BlockSpec((N_TILE, M), lambda i: (i, 0))],
        out_specs=[
            pl.BlockSpec((N_TILE, KNN), lambda i: (i, 0)),
            pl.BlockSpec((N_TILE, KNN), lambda i: (i, 0)),
        ],
        out_shape=[
            jax.ShapeDtypeStruct((N, KNN), jnp.float32),
            jax.ShapeDtypeStruct((N, KNN), jnp.int32),
        ],
    )(dist)
    return (knn_dist, knn_ind)


# speedup vs baseline: 18.7466x; 1.1107x over previous
import jax
import jax.numpy as jnp
from jax.experimental import pallas as pl

T = 32
N = 512
M = 2048
TOPK_TIME = 8
KNN = 16
N_TILE = 128
M_TILE = 1024


_SORT8_NET = [(0, 1), (2, 3), (4, 5), (6, 7),
              (0, 2), (1, 3), (4, 6), (5, 7),
              (1, 2), (5, 6), (0, 4), (3, 7),
              (1, 5), (2, 6),
              (1, 4), (3, 6),
              (2, 4), (3, 5),
              (3, 4)]

_BITONIC8_NET = [(0, 4), (1, 5), (2, 6), (3, 7),
                 (0, 2), (1, 3), (4, 6), (5, 7),
                 (0, 1), (2, 3), (4, 5), (6, 7)]


def _apply_net(vs, net):
    vs = list(vs)
    for i, j in net:
        hi = jnp.maximum(vs[i], vs[j])
        lo = jnp.minimum(vs[i], vs[j])
        vs[i], vs[j] = hi, lo
    return vs


def _dist_kernel(q_ref, b_ref, dist_ref):
    q = q_ref[...]
    b = b_ref[...]
    run = None
    n_groups = T // TOPK_TIME
    for g in range(n_groups):
        d2s = []
        for tt in range(TOPK_TIME):
            t = g * TOPK_TIME + tt
            qt = q[:, t, :]
            bt = b[t]
            q2 = 0.25 * jnp.sum(qt * qt, axis=1, keepdims=True)
            b2 = jnp.sum(bt * bt, axis=0, keepdims=True)
            cross = jax.lax.dot_general(
                qt, bt, (((1,), (0,)), ((), ())),
                preferred_element_type=jnp.float32)
            d2s.append((q2 + b2) + cross)
        s = _apply_net(d2s, _SORT8_NET)
        if run is None:
            run = s
        elif g < n_groups - 1:
            tops = [jnp.maximum(run[i], s[TOPK_TIME - 1 - i])
                    for i in range(TOPK_TIME)]
            run = _apply_net(tops, _BITONIC8_NET)
        else:
            tops = [jnp.maximum(run[i], s[TOPK_TIME - 1 - i])
                    for i in range(TOPK_TIME)]
            d2_sel = tops[0]
            for i in range(1, TOPK_TIME):
                d2_sel = jnp.minimum(d2_sel, tops[i])
    dist_ref[...] = jnp.sqrt(jnp.clip(d2_sel, 0.0, None) + 1e-12)


def _topk_kernel(dist_ref, knn_dist_ref, knn_ind_ref):
    work = dist_ref[...]
    iota = jax.lax.broadcasted_iota(jnp.int32, (N_TILE, M), 1)
    big_i = jnp.int32(M)
    pos_inf = jnp.float32(jnp.inf)
    for k in range(KNN):
        mv = jnp.min(work, axis=1, keepdims=True)
        idx = jnp.min(jnp.where(work == mv, iota, big_i), axis=1,
                      keepdims=True)
        knn_dist_ref[:, k] = mv[:, 0]
        knn_ind_ref[:, k] = idx[:, 0]
        work = jnp.where(iota == idx, pos_inf, work)


@jax.jit
def kernel(q_curve_xyz, b_curve_xyz):
    q_r = -2.0 * jnp.transpose(q_curve_xyz, (1, 0, 2))
    b_r = jnp.transpose(b_curve_xyz, (0, 2, 1))

    dist = pl.pallas_call(
        _dist_kernel,
        grid=(N // N_TILE, M // M_TILE),
        in_specs=[
            pl.BlockSpec((N_TILE, T, 3), lambda i, j: (i, 0, 0)),
            pl.BlockSpec((T, 3, M_TILE), lambda i, j: (0, 0, j)),
        ],
        out_specs=pl.BlockSpec((N_TILE, M_TILE), lambda i, j: (i, j)),
        out_shape=jax.ShapeDtypeStruct((N, M), jnp.float32),
    )(q_r, b_r)

    knn_dist, knn_ind = pl.pallas_call(
        _topk_kernel,
        grid=(N // N_TILE,),
        in_specs=[pl.BlockSpec((N_TILE, M), lambda i: (i, 0))],
        out_specs=[
            pl.BlockSpec((N_TILE, KNN), lambda i: (i, 0)),
            pl.BlockSpec((N_TILE, KNN), lambda i: (i, 0)),
        ],
        out_shape=[
            jax.ShapeDtypeStruct((N, KNN), jnp.float32),
            jax.ShapeDtypeStruct((N, KNN), jnp.int32),
        ],
    )(dist)
    return (knn_dist, knn_ind)
